# baseline (speedup 1.0000x reference)
import functools

import jax
import jax.numpy as jnp
from jax import lax
from jax.experimental import pallas as pl
from jax.experimental.pallas import tpu as pltpu
from jax.experimental.pallas import tpu_sc as plsc

_B, _S, _D = 4, 8192, 1024
_NC, _NS = 2, 16
_NW = _NC * _NS
_ROWS_PER_W = _S // _NW
_CH = 16
_CHW = _CH * _D
_NCH = _ROWS_PER_W // _CH


def _sc_body(x_hbm, pe_hbm, out_hbm, pe_v, x_v):
    wid = lax.axis_index("s") * _NC + lax.axis_index("c")
    base = wid * (_ROWS_PER_W * _D)

    def chunk_body(ci, carry):
        off = base + ci * _CHW
        pltpu.sync_copy(pe_hbm.at[pl.ds(off, _CHW)], pe_v)
        for b in range(_B):
            xoff = b * (_S * _D) + off
            pltpu.sync_copy(x_hbm.at[pl.ds(xoff, _CHW)], x_v)

            def add_body(i):
                x_v[pl.ds(i, 16)] = x_v[pl.ds(i, 16)] + pe_v[pl.ds(i, 16)]

            plsc.parallel_loop(0, _CHW, 16, unroll=8)(add_body)
            pltpu.sync_copy(x_v, out_hbm.at[pl.ds(xoff, _CHW)])
        return carry

    lax.fori_loop(0, _NCH, chunk_body, 0)


def kernel(x, pe):
    B, S, D = x.shape
    mesh = plsc.VectorSubcoreMesh(core_axis_name="c", subcore_axis_name="s")
    out_flat = pl.kernel(
        _sc_body,
        out_type=jax.ShapeDtypeStruct((B * S * D,), jnp.float32),
        mesh=mesh,
        scratch_types=[
            pltpu.VMEM((_CHW,), jnp.float32),
            pltpu.VMEM((_CHW,), jnp.float32),
        ],
    )(x.reshape(-1), pe.reshape(-1))
    return out_flat.reshape(B, S, D)


# speedup vs baseline: 1.3013x; 1.3013x over previous
import jax
import jax.numpy as jnp
from jax import lax
from jax.experimental import pallas as pl
from jax.experimental.pallas import tpu as pltpu
from jax.experimental.pallas import tpu_sc as plsc

_B, _S, _D = 4, 8192, 1024
_NC, _NS = 2, 16
_NW = _NC * _NS
_ROWS_PER_W = _S // _NW
_CH = 16
_CHW = _CH * _D
_NCH = _ROWS_PER_W // _CH
_SD = _S * _D


def _sc_body(x_hbm, pe_hbm, out_hbm,
             xin0, xin1, ob0, ob1, pev0, pev1,
             ld0, ld1, st0, st1, ps0, ps1):
    xin, ob, pev = [xin0, xin1], [ob0, ob1], [pev0, pev1]
    ld, st, ps = [ld0, ld1], [st0, st1], [ps0, ps1]

    wid = lax.axis_index("s") * _NC + lax.axis_index("c")
    base = wid * (_ROWS_PER_W * _D)

    pltpu.async_copy(x_hbm.at[pl.ds(base, _CHW)], xin[0], ld[0])
    pltpu.async_copy(x_hbm.at[pl.ds(base + _SD, _CHW)], xin[1], ld[1])
    pltpu.async_copy(pe_hbm.at[pl.ds(base, _CHW)], pev[0], ps[0])
    pltpu.async_copy(pe_hbm.at[pl.ds(base + _CHW, _CHW)], pev[1], ps[1])
    pltpu.async_copy(ob[0], out_hbm.at[pl.ds(base, _CHW)], st[0])
    pltpu.async_copy(ob[1], out_hbm.at[pl.ds(base + _SD, _CHW)], st[1])

    def outer(k, carry):
        ci0 = 2 * k
        for dci in range(2):
            ci = ci0 + dci
            q = dci
            off = base + ci * _CHW
            for b in range(_B):
                p = b % 2
                xoff = off + b * _SD
                pltpu.make_async_copy(
                    x_hbm.at[pl.ds(0, _CHW)], xin[p], ld[p]).wait()
                if b == 0:
                    pltpu.make_async_copy(
                        pe_hbm.at[pl.ds(0, _CHW)], pev[q], ps[q]).wait()
                pltpu.make_async_copy(
                    ob[p], out_hbm.at[pl.ds(0, _CHW)], st[p]).wait()

                xin_p, ob_p, pev_q = xin[p], ob[p], pev[q]

                @plsc.parallel_loop(0, _CHW, 16, unroll=8)
                def add_body(i):
                    ob_p[pl.ds(i, 16)] = xin_p[pl.ds(i, 16)] + pev_q[pl.ds(i, 16)]

                pltpu.async_copy(ob[p], out_hbm.at[pl.ds(xoff, _CHW)], st[p])

                if b < 2:
                    pltpu.async_copy(
                        x_hbm.at[pl.ds(xoff + 2 * _SD, _CHW)], xin[p], ld[p])
                else:
                    @pl.when(ci < _NCH - 1)
                    def _():
                        pltpu.async_copy(
                            x_hbm.at[pl.ds(off + _CHW + (b - 2) * _SD, _CHW)],
                            xin[p], ld[p])
                if b == 3:
                    @pl.when(ci < _NCH - 2)
                    def _():
                        pltpu.async_copy(
                            pe_hbm.at[pl.ds(off + 2 * _CHW, _CHW)], pev[q], ps[q])
        return carry

    lax.fori_loop(0, _NCH // 2, outer, 0)

    pltpu.make_async_copy(ob[0], out_hbm.at[pl.ds(0, _CHW)], st[0]).wait()
    pltpu.make_async_copy(ob[1], out_hbm.at[pl.ds(0, _CHW)], st[1]).wait()


def kernel(x, pe):
    B, S, D = x.shape
    mesh = plsc.VectorSubcoreMesh(core_axis_name="c", subcore_axis_name="s")
    out_flat = pl.kernel(
        _sc_body,
        out_type=jax.ShapeDtypeStruct((B * S * D,), jnp.float32),
        mesh=mesh,
        scratch_types=(
            [pltpu.VMEM((_CHW,), jnp.float32) for _ in range(6)]
            + [pltpu.SemaphoreType.DMA for _ in range(6)]
        ),
    )(x.reshape(-1), pe.reshape(-1))
    return out_flat.reshape(B, S, D)


# speedup vs baseline: 3.3696x; 2.5895x over previous
import jax
import jax.numpy as jnp
from jax import lax
from jax.experimental import pallas as pl
from jax.experimental.pallas import tpu as pltpu
from jax.experimental.pallas import tpu_sc as plsc

_B, _S, _D = 4, 8192, 1024
_NC, _NS = 2, 16
_NW = _NC * _NS
_ROWS_PER_W = _S // _NW
_CH = 16
_NCH = _ROWS_PER_W // _CH


def _sc_body(x_hbm, pe_hbm, out_hbm,
             xin0, xin1, ob0, ob1, pev0, pev1,
             ld0, ld1, st0, st1, ps0, ps1):
    xin, ob, pev = [xin0, xin1], [ob0, ob1], [pev0, pev1]
    ld, st, ps = [ld0, ld1], [st0, st1], [ps0, ps1]

    wid = lax.axis_index("s") * _NC + lax.axis_index("c")
    base = wid * _ROWS_PER_W

    pltpu.async_copy(x_hbm.at[pl.ds(base, _CH)], xin[0], ld[0])
    pltpu.async_copy(x_hbm.at[pl.ds(base + _S, _CH)], xin[1], ld[1])
    pltpu.async_copy(pe_hbm.at[pl.ds(base, _CH)], pev[0], ps[0])
    pltpu.async_copy(pe_hbm.at[pl.ds(base + _CH, _CH)], pev[1], ps[1])
    pltpu.async_copy(ob[0], out_hbm.at[pl.ds(base, _CH)], st[0])
    pltpu.async_copy(ob[1], out_hbm.at[pl.ds(base + _S, _CH)], st[1])

    def outer(k, carry):
        ci0 = 2 * k
        for dci in range(2):
            ci = ci0 + dci
            q = dci
            row = base + ci * _CH
            for b in range(_B):
                p = b % 2
                xrow = row + b * _S
                pltpu.make_async_copy(
                    x_hbm.at[pl.ds(0, _CH)], xin[p], ld[p]).wait()
                if b == 0:
                    pltpu.make_async_copy(
                        pe_hbm.at[pl.ds(0, _CH)], pev[q], ps[q]).wait()
                pltpu.make_async_copy(
                    ob[p], out_hbm.at[pl.ds(0, _CH)], st[p]).wait()

                xin_p, ob_p, pev_q = xin[p], ob[p], pev[q]

                for r in range(_CH):
                    @plsc.parallel_loop(0, _D, 16, unroll=8)
                    def add_body(c, _r=r):
                        ob_p[_r, pl.ds(c, 16)] = (
                            xin_p[_r, pl.ds(c, 16)] + pev_q[_r, pl.ds(c, 16)])

                pltpu.async_copy(ob[p], out_hbm.at[pl.ds(xrow, _CH)], st[p])

                if b < 2:
                    pltpu.async_copy(
                        x_hbm.at[pl.ds(xrow + 2 * _S, _CH)], xin[p], ld[p])
                else:
                    @pl.when(ci < _NCH - 1)
                    def _():
                        pltpu.async_copy(
                            x_hbm.at[pl.ds(row + _CH + (b - 2) * _S, _CH)],
                            xin[p], ld[p])
                if b == 3:
                    @pl.when(ci < _NCH - 2)
                    def _():
                        pltpu.async_copy(
                            pe_hbm.at[pl.ds(row + 2 * _CH, _CH)], pev[q], ps[q])
        return carry

    lax.fori_loop(0, _NCH // 2, outer, 0)

    pltpu.make_async_copy(ob[0], out_hbm.at[pl.ds(0, _CH)], st[0]).wait()
    pltpu.make_async_copy(ob[1], out_hbm.at[pl.ds(0, _CH)], st[1]).wait()


def kernel(x, pe):
    B, S, D = x.shape
    mesh = plsc.VectorSubcoreMesh(core_axis_name="c", subcore_axis_name="s")
    out2d = pl.kernel(
        _sc_body,
        out_type=jax.ShapeDtypeStruct((B * S, D), jnp.float32),
        mesh=mesh,
        scratch_types=(
            [pltpu.VMEM((_CH, _D), jnp.float32) for _ in range(6)]
            + [pltpu.SemaphoreType.DMA for _ in range(6)]
        ),
        compiler_params=pltpu.CompilerParams(use_tc_tiling_on_sc=True),
    )(x.reshape(B * S, D), pe)
    return out2d.reshape(B, S, D)


# speedup vs baseline: 3.9789x; 1.1808x over previous
import jax
import jax.numpy as jnp
from jax import lax
from jax.experimental import pallas as pl
from jax.experimental.pallas import tpu as pltpu
from jax.experimental.pallas import tpu_sc as plsc

_B, _S, _D = 4, 8192, 1024
_NC, _NS = 2, 16
_NW = _NC * _NS
_ROWS_PER_W = _S // _NW
_CH = 8
_NCH = _ROWS_PER_W // _CH


def _sc_body(x_hbm, pe_hbm, out_hbm, *refs):
    xin = [[refs[2 * b + p] for p in range(2)] for b in range(_B)]
    ob = list(refs[8:12])
    pev = list(refs[12:14])
    ld = [[refs[14 + 2 * b + p] for p in range(2)] for b in range(_B)]
    st = list(refs[22:26])
    ps = list(refs[26:28])

    wid = lax.axis_index("s") * _NC + lax.axis_index("c")
    base = wid * _ROWS_PER_W

    for c in range(2):
        pltpu.async_copy(pe_hbm.at[pl.ds(base + c * _CH, _CH)], pev[c], ps[c])
        for b in range(_B):
            pltpu.async_copy(
                x_hbm.at[pl.ds(base + b * _S + c * _CH, _CH)], xin[b][c],
                ld[b][c])
    for b in range(_B):
        pltpu.async_copy(ob[b], out_hbm.at[pl.ds(base + b * _S, _CH)], st[b])

    def outer(k, carry):
        ci0 = 2 * k
        for p in range(2):
            ci = ci0 + p
            row = base + ci * _CH
            pltpu.make_async_copy(
                pe_hbm.at[pl.ds(0, _CH)], pev[p], ps[p]).wait()
            for b in range(_B):
                pltpu.make_async_copy(
                    x_hbm.at[pl.ds(0, _CH)], xin[b][p], ld[b][p]).wait()
                pltpu.make_async_copy(
                    ob[b], out_hbm.at[pl.ds(0, _CH)], st[b]).wait()

            xin_p = [xin[b][p] for b in range(_B)]
            pev_p = pev[p]

            for r in range(_CH):
                @plsc.parallel_loop(0, _D, 16, unroll=4)
                def add_body(c, _r=r):
                    pe16 = pev_p[_r, pl.ds(c, 16)]
                    for b in range(_B):
                        ob[b][_r, pl.ds(c, 16)] = (
                            xin_p[b][_r, pl.ds(c, 16)] + pe16)

            for b in range(_B):
                pltpu.async_copy(
                    ob[b], out_hbm.at[pl.ds(row + b * _S, _CH)], st[b])

            @pl.when(ci < _NCH - 2)
            def _():
                pltpu.async_copy(
                    pe_hbm.at[pl.ds(row + 2 * _CH, _CH)], pev[p], ps[p])
                for b in range(_B):
                    pltpu.async_copy(
                        x_hbm.at[pl.ds(row + b * _S + 2 * _CH, _CH)],
                        xin[b][p], ld[b][p])
        return carry

    lax.fori_loop(0, _NCH // 2, outer, 0)

    for b in range(_B):
        pltpu.make_async_copy(ob[b], out_hbm.at[pl.ds(0, _CH)], st[b]).wait()


def kernel(x, pe):
    B, S, D = x.shape
    mesh = plsc.VectorSubcoreMesh(core_axis_name="c", subcore_axis_name="s")
    out2d = pl.kernel(
        _sc_body,
        out_type=jax.ShapeDtypeStruct((B * S, D), jnp.float32),
        mesh=mesh,
        scratch_types=(
            [pltpu.VMEM((_CH, _D), jnp.float32) for _ in range(14)]
            + [pltpu.SemaphoreType.DMA for _ in range(14)]
        ),
        compiler_params=pltpu.CompilerParams(use_tc_tiling_on_sc=True),
    )(x.reshape(B * S, D), pe)
    return out2d.reshape(B, S, D)
